# Initial kernel scaffold; baseline (speedup 1.0000x reference)
import jax
import jax.numpy as jnp
from jax.experimental import pallas as pl


def kernel(x):
    raise NotImplementedError("write your pallas kernel here")


# speedup vs baseline: 8.7100x; 8.7100x over previous
import jax
import jax.numpy as jnp
from jax.experimental import pallas as pl
from jax.experimental.pallas import tpu as pltpu

_DENSITY = 0.05
_TOP_IGNORE = 7
_INT_MIN = -2147483648
_NT = 8


def _sel_body(x_ref, thr_ref, cmean_ref, ks_s, acc_lo, acc_hi, cnt_lo, cnt_hi):
    b = pl.program_id(0)
    t = pl.program_id(1)
    n_rows, tile = x_ref.shape
    n_total = tile * _NT
    k_lo = int(n_total * _DENSITY) + _TOP_IGNORE
    k_hi = _TOP_IGNORE

    @pl.when(b == 0)
    def _load():
        x = x_ref[...]
        u = jax.lax.bitcast_convert_type(x, jnp.int32)
        ks_s[:, pl.ds(t * tile, tile)] = jnp.where(u >= 0, u, u ^ jnp.int32(0x7FFFFFFF))

        @pl.when(t == 0)
        def _init():
            acc_lo[...] = jnp.zeros_like(acc_lo)
            acc_hi[...] = jnp.zeros_like(acc_hi)

    @pl.when((b >= 1) & (b <= 32))
    def _bit():
        bit = jax.lax.shift_left(jnp.int32(1), jnp.int32(32) - b)
        ks = ks_s[:, pl.ds(t * tile, tile)]

        def count(acc_ref):
            t_u = acc_ref[...] | bit
            t_s = t_u ^ jnp.int32(_INT_MIN)
            return jnp.sum((ks >= t_s).astype(jnp.int32), axis=1, keepdims=True)

        c_lo = count(acc_lo)
        c_hi = count(acc_hi)

        @pl.when(t == 0)
        def _reset():
            cnt_lo[...] = c_lo
            cnt_hi[...] = c_hi

        @pl.when(t > 0)
        def _accum():
            cnt_lo[...] += c_lo
            cnt_hi[...] += c_hi

        @pl.when(t == _NT - 1)
        def _update():
            acc_lo[...] = jnp.where(cnt_lo[...] >= k_lo, acc_lo[...] | bit, acc_lo[...])
            acc_hi[...] = jnp.where(cnt_hi[...] >= k_hi, acc_hi[...] | bit, acc_hi[...])

    @pl.when(b == 33)
    def _decode():
        def dec(key):
            bits = jnp.where(key >= 0, key, key ^ jnp.int32(0x7FFFFFFF))
            return jax.lax.bitcast_convert_type(bits, jnp.float32)

        thr_ref[...] = jnp.concatenate(
            [
                dec(acc_lo[...] ^ jnp.int32(_INT_MIN)),
                dec(acc_hi[...] ^ jnp.int32(_INT_MIN)),
                jnp.zeros((n_rows, 126), jnp.float32),
            ],
            axis=1,
        )
        xt = dec(ks_s[:, pl.ds(t * tile, tile)])
        cmean_ref[...] = jnp.mean(xt, axis=0, keepdims=True)


def _combine_body(x_ref, thr_ref, cmean_ref, o_ref):
    x = x_ref[...]
    t_lo = thr_ref[:, 0:1]
    t_hi = thr_ref[:, 1:2]
    below = x <= t_hi
    inband = below & (x >= t_lo)
    o_ref[...] = jnp.where(inband, x, 0.0) - cmean_ref[...] * below.astype(jnp.float32)


def kernel(x):
    n_rows, n = x.shape
    tile = n // _NT
    thr, cmean = pl.pallas_call(
        _sel_body,
        grid=(34, _NT),
        in_specs=[pl.BlockSpec((n_rows, tile), lambda b, t: (0, jnp.where(b == 0, t, 0)))],
        out_specs=[
            pl.BlockSpec((n_rows, 128), lambda b, t: (0, 0)),
            pl.BlockSpec((1, tile), lambda b, t: (0, jnp.where(b == 33, t, 0))),
        ],
        out_shape=[
            jax.ShapeDtypeStruct((n_rows, 128), jnp.float32),
            jax.ShapeDtypeStruct((1, n), jnp.float32),
        ],
        scratch_shapes=[
            pltpu.VMEM((n_rows, n), jnp.int32),
            pltpu.VMEM((n_rows, 1), jnp.int32),
            pltpu.VMEM((n_rows, 1), jnp.int32),
            pltpu.VMEM((n_rows, 1), jnp.int32),
            pltpu.VMEM((n_rows, 1), jnp.int32),
        ],
    )(x)

    ct = n // _NT
    return pl.pallas_call(
        _combine_body,
        grid=(_NT,),
        in_specs=[
            pl.BlockSpec((n_rows, ct), lambda t: (0, t)),
            pl.BlockSpec((n_rows, 128), lambda t: (0, 0)),
            pl.BlockSpec((1, ct), lambda t: (0, t)),
        ],
        out_specs=pl.BlockSpec((n_rows, ct), lambda t: (0, t)),
        out_shape=jax.ShapeDtypeStruct((n_rows, n), jnp.float32),
    )(x, thr, cmean)
